# baseline (speedup 1.0000x reference)
import functools
import jax
import jax.numpy as jnp
from jax import lax
from jax.experimental import pallas as pl
from jax.experimental.pallas import tpu as pltpu
from jax.experimental.pallas import tpu_sc as plsc

N = 2048
NW = 32
JPW = N // NW
NCH = N // 16


def _tc_pre(h_ref, p_ref, feat_ref):
    h0 = h_ref[0:1, :]
    h1 = h_ref[1:2, :]
    p = p_ref
    hh0 = h0 * p[0, 0] + h1 * p[0, 1]
    hh1 = h0 * p[0, 2] + h1 * p[0, 3]
    s = hh0 * p[0, 4] + hh1 * p[0, 5]
    d = hh0 * p[0, 6] + hh1 * p[0, 7]
    m1 = jnp.max(s)
    e1 = jnp.exp(s - m1)
    e02 = jnp.exp(0.2 * (s - m1))
    feat_ref[0:1, :] = s
    feat_ref[1:2, :] = -d
    feat_ref[2:3, :] = e02
    feat_ref[3:4, :] = e02 * hh0
    feat_ref[4:5, :] = e02 * hh1
    feat_ref[5:6, :] = e1
    feat_ref[6:7, :] = e1 * hh0
    feat_ref[7:8, :] = e1 * hh1


def _sc_edge(feat_hbm, out_hbm, s_v, t_v, v0_v, v1_v, v2_v, v3_v, v4_v, v5_v,
             r0_v, r1_v, r2_v, r3_v, r4_v, r5_v):
    cid = lax.axis_index("c")
    sid = lax.axis_index("s")
    wid = sid * 2 + cid
    base = wid * JPW
    rows = (s_v, t_v, v0_v, v1_v, v2_v, v3_v, v4_v, v5_v)
    for r, rv in enumerate(rows):
        pltpu.sync_copy(feat_hbm.at[pl.ds(r * N, N)], rv)

    lanes = lax.iota(jnp.int32, 16)

    def j_group(g, _):
        def j_body(jj, rvs):
            j = base + g * 16 + jj
            tj = plsc.load_gather(t_v, [jnp.full((16,), j, jnp.int32)])

            def chunk(c, accs):
                sl = pl.ds(c * 16, 16)
                m = s_v[sl] <= tj
                z = jnp.zeros((16,), jnp.float32)
                return (accs[0] + jnp.where(m, v0_v[sl], z),
                        accs[1] + jnp.where(m, v1_v[sl], z),
                        accs[2] + jnp.where(m, v2_v[sl], z),
                        accs[3] + jnp.where(m, v3_v[sl], z),
                        accs[4] + jnp.where(m, v4_v[sl], z),
                        accs[5] + jnp.where(m, v5_v[sl], z))

            z6 = tuple(jnp.zeros((16,), jnp.float32) for _ in range(6))
            accs = lax.fori_loop(0, NCH, chunk, z6)
            sel = lanes == jj
            return tuple(
                jnp.where(sel, jnp.sum(a, axis=0), rv)
                for a, rv in zip(accs, rvs))

        z6 = tuple(jnp.zeros((16,), jnp.float32) for _ in range(6))
        out16 = lax.fori_loop(0, 16, j_body, z6)
        res = (r0_v, r1_v, r2_v, r3_v, r4_v, r5_v)
        for rv, o in zip(res, out16):
            rv[pl.ds(g * 16, 16)] = o
        return 0

    lax.fori_loop(0, JPW // 16, j_group, 0)
    for r, rv in enumerate((r0_v, r1_v, r2_v, r3_v, r4_v, r5_v)):
        pltpu.sync_copy(rv, out_hbm.at[pl.ds(r * N + base, JPW)])


def _tc_post(feat_ref, w_ref, p_ref, out_ref):
    s = feat_ref[0:1, :]
    d = -feat_ref[1:2, :]
    m1 = jnp.max(s)
    T1 = jnp.sum(feat_ref[5:6, :])
    Tq0 = jnp.sum(feat_ref[6:7, :])
    Tq1 = jnp.sum(feat_ref[7:8, :])
    b2 = d + m1
    b1 = 0.2 * b2
    L = jnp.maximum(b1, b2)
    f1 = jnp.exp(b1 - L)
    f2 = jnp.exp(b2 - L)
    den = f1 * w_ref[0:1, :] + f2 * (T1 - w_ref[3:4, :])
    num0 = f1 * w_ref[1:2, :] + f2 * (Tq0 - w_ref[4:5, :])
    num1 = f1 * w_ref[2:3, :] + f2 * (Tq1 - w_ref[5:6, :])
    out_ref[0:1, :] = num0 / den + p_ref[0, 8]
    out_ref[1:2, :] = num1 / den + p_ref[0, 9]


_sc_call = pl.kernel(
    _sc_edge,
    out_type=jax.ShapeDtypeStruct((6 * N,), jnp.float32),
    mesh=plsc.VectorSubcoreMesh(core_axis_name="c", subcore_axis_name="s",
                                num_cores=2, num_subcores=16),
    scratch_types=[pltpu.VMEM((N,), jnp.float32)] * 8
    + [pltpu.VMEM((JPW,), jnp.float32)] * 6,
    compiler_params=pltpu.CompilerParams(needs_layout_passes=False),
)


def _layer(h2, params):
    feat = pl.pallas_call(
        _tc_pre,
        out_shape=jax.ShapeDtypeStruct((8, N), jnp.float32),
    )(h2, params)
    w = _sc_call(feat.reshape(8 * N)).reshape(6, N)
    return pl.pallas_call(
        _tc_post,
        out_shape=jax.ShapeDtypeStruct((2, N), jnp.float32),
    )(feat, w, params)


def _pack_params(lin_w, att_src, att_dst, bias):
    v = jnp.concatenate([
        lin_w[0].reshape(2), lin_w[1].reshape(2),
        att_src.reshape(2), att_dst.reshape(2), bias.reshape(2),
        jnp.zeros((6,), jnp.float32)])
    return v.reshape(1, 16)


@jax.jit
def kernel(x, lin_weight_0, src_weight_0, dst_weight_0, bias_weight_0,
           lin_weight_1, src_weight_1, dst_weight_1, bias_weight_1):
    xpos = (jnp.arange(N, dtype=jnp.float32) - N / 2).reshape(1, N)
    h2 = jnp.concatenate([x.reshape(1, N), xpos], axis=0)
    h2 = _layer(h2, _pack_params(lin_weight_0, src_weight_0, dst_weight_0,
                                 bias_weight_0))
    h2 = _layer(h2, _pack_params(lin_weight_1, src_weight_1, dst_weight_1,
                                 bias_weight_1))
    return h2.T


# speedup vs baseline: 6.9089x; 6.9089x over previous
import jax
import jax.numpy as jnp
from jax import lax
from jax.experimental import pallas as pl

N = 2048


def _masked_sum(v, mask_b):
    vh = v.astype(jnp.bfloat16)
    r1 = v - vh.astype(jnp.float32)
    vm = r1.astype(jnp.bfloat16)
    vl = (r1 - vm.astype(jnp.float32)).astype(jnp.bfloat16)
    v9 = jnp.concatenate([vh, vm, vl], axis=0)
    w9 = jnp.dot(v9, mask_b, preferred_element_type=jnp.float32)
    return w9[0:3, :] + w9[3:6, :] + w9[6:9, :]


def _layer(h0, h1, p):
    hh0 = h0 * p[0, 0] + h1 * p[0, 1]
    hh1 = h0 * p[0, 2] + h1 * p[0, 3]
    s = hh0 * p[0, 4] + hh1 * p[0, 5]
    d = hh0 * p[0, 6] + hh1 * p[0, 7]
    m1 = jnp.max(s)
    e1 = jnp.exp(s - m1)
    e02 = jnp.exp(0.2 * (s - m1))
    q0 = e1 * hh0
    q1 = e1 * hh1
    ones = jnp.ones_like(s)
    A = jnp.concatenate([s, ones], axis=0)
    B = jnp.concatenate([ones, d], axis=0)
    Z = lax.dot_general(A, B, (((0,), (0,)), ((), ())),
                        preferred_element_type=jnp.float32)
    Mt = jnp.where(Z <= 0.0, 1.0, 0.0).astype(jnp.bfloat16)
    Mt2 = jnp.where(Z > 0.0, 1.0, 0.0).astype(jnp.bfloat16)
    Vn = jnp.concatenate([e02, e02 * hh0, e02 * hh1], axis=0)
    Vp = jnp.concatenate([e1, q0, q1], axis=0)
    Wn = _masked_sum(Vn, Mt)
    Wp = _masked_sum(Vp, Mt2)
    b2 = d + m1
    b1 = 0.2 * b2
    L = jnp.maximum(b1, b2)
    f1 = jnp.exp(b1 - L)
    f2 = jnp.exp(b2 - L)
    den = f1 * Wn[0:1, :] + f2 * Wp[0:1, :]
    o0 = (f1 * Wn[1:2, :] + f2 * Wp[1:2, :]) / den + p[0, 8]
    o1 = (f1 * Wn[2:3, :] + f2 * Wp[2:3, :]) / den + p[0, 9]
    return o0, o1


def _gnn_kernel(x_ref, p0_ref, p1_ref, out_ref):
    x0 = x_ref[...]
    xpos = (lax.broadcasted_iota(jnp.int32, (1, N), 1)
            .astype(jnp.float32) - N / 2)
    o0, o1 = _layer(x0, xpos, p0_ref[...])
    o0, o1 = _layer(o0, o1, p1_ref[...])
    out_ref[0:1, :] = o0
    out_ref[1:2, :] = o1


def _pack_params(lin_w, att_src, att_dst, bias):
    v = jnp.concatenate([
        lin_w[0].reshape(2), lin_w[1].reshape(2),
        att_src.reshape(2), att_dst.reshape(2), bias.reshape(2),
        jnp.zeros((6,), jnp.float32)])
    return v.reshape(1, 16)


@jax.jit
def kernel(x, lin_weight_0, src_weight_0, dst_weight_0, bias_weight_0,
           lin_weight_1, src_weight_1, dst_weight_1, bias_weight_1):
    p0 = _pack_params(lin_weight_0, src_weight_0, dst_weight_0, bias_weight_0)
    p1 = _pack_params(lin_weight_1, src_weight_1, dst_weight_1, bias_weight_1)
    out = pl.pallas_call(
        _gnn_kernel,
        out_shape=jax.ShapeDtypeStruct((2, N), jnp.float32),
    )(x.reshape(1, N), p0, p1)
    return out.T
